# baseline (speedup 1.0000x reference)
import functools

import jax
import jax.numpy as jnp
from jax import lax
from jax.experimental import pallas as pl
from jax.experimental.pallas import tpu as pltpu
from jax.experimental.pallas import tpu_sc as plsc

_B, _S, _D, _T = 16, 512, 512, 2048
_L = 16
_NW = 32
_ROWS_W = (_B * _T) // _NW
_CHUNK = 32
_NCHUNK = _ROWS_W // _CHUNK
_ZROWS = 32
_NB = 6
_GF = 3


def _build_expand():
    mesh = plsc.VectorSubcoreMesh(core_axis_name="c", subcore_axis_name="s")

    @functools.partial(
        pl.kernel,
        mesh=mesh,
        compiler_params=pltpu.CompilerParams(needs_layout_passes=False),
        out_type=(
            jax.ShapeDtypeStruct((_B * _T, _D), jnp.float32),
            jax.ShapeDtypeStruct((_B,), jnp.int32),
        ),
        scratch_types=[
            pltpu.VMEM((_S,), jnp.int32),
            pltpu.VMEM((_S,), jnp.int32),
            pltpu.VMEM((_ROWS_W,), jnp.int32),
            *[pltpu.VMEM((_CHUNK, _D), jnp.float32) for _ in range(_NB)],
            pltpu.VMEM((_ZROWS, _D), jnp.float32),
            pltpu.VMEM((_B * _S,), jnp.int32),
            pltpu.VMEM((_B,), jnp.int32),
            pltpu.VMEM((_L,), jnp.int32),
            *[pltpu.SemaphoreType.DMA for _ in range(_NB)],
            *[pltpu.SemaphoreType.DMA for _ in range(_NB)],
            pltpu.SemaphoreType.DMA,
        ],
    )
    def expand(dur_hbm, xflat_hbm, zc_hbm, ml_hbm, out_hbm, len_hbm,
               *refs):
        bufs = refs[3:3 + _NB]
        (dur_v, cum_v, gidx_v) = refs[:3]
        (zbuf, dur_all_v, len_v, ml_v) = refs[3 + _NB:7 + _NB]
        gsems = refs[7 + _NB:7 + 2 * _NB]
        wsems = refs[7 + 2 * _NB:7 + 3 * _NB]
        zsem = refs[7 + 3 * _NB]
        cid = lax.axis_index("c")
        sid = lax.axis_index("s")
        wid = cid * (_NW // 2) + sid
        b = wid // 2
        t0 = (wid % 2) * (_T // 2)

        pltpu.sync_copy(dur_hbm.at[pl.ds(b * _S, _S)], dur_v)
        pltpu.sync_copy(ml_hbm, ml_v)
        pltpu.sync_copy(zc_hbm, zbuf)
        max_len = ml_v[...][0]

        def cum_step(i, carry):
            v = jnp.clip(dur_v[pl.ds(i * _L, _L)], 0, 300)
            cum_v[pl.ds(i * _L, _L)] = plsc.cumsum(v) + carry
            return carry + jnp.sum(v)

        total = lax.fori_loop(0, _S // _L, cum_step, jnp.int32(0))
        len_b = jnp.minimum(total, max_len)
        len_local = jnp.clip(len_b - t0, 0, _ROWS_W)

        def idx_step(i, _):
            t = t0 + i * _L + lax.iota(jnp.int32, _L)
            j = jnp.zeros((_L,), jnp.int32)
            for s in (256, 128, 64, 32, 16, 8, 4, 2, 1):
                m = j + s
                probe = plsc.load_gather(cum_v, [jnp.minimum(m - 1, _S - 1)])
                j = jnp.where(probe <= t, m, j)
            tok = jnp.where(t < len_b, jnp.minimum(j, _S - 1), 0)
            gidx_v[pl.ds(i * _L, _L)] = b * _S + tok
            return 0

        lax.fori_loop(0, _ROWS_W // _L, idx_step, 0)

        row0 = wid * _ROWS_W

        def valid(c):
            return c * _CHUNK < len_local

        def gstart(c):
            @pl.when(valid(c))
            def _():
                pltpu.async_copy(
                    xflat_hbm.at[gidx_v.at[pl.ds(c * _CHUNK, _CHUNK)]],
                    bufs[c % _NB], gsems[c % _NB])

        def gwait(c):
            @pl.when(valid(c))
            def _():
                pltpu.make_async_copy(
                    xflat_hbm.at[pl.ds(0, _CHUNK)], bufs[c % _NB],
                    gsems[c % _NB]).wait()

        def wstart(c):
            @pl.when(valid(c))
            def _():
                pltpu.async_copy(
                    bufs[c % _NB], out_hbm.at[pl.ds(row0 + c * _CHUNK, _CHUNK)],
                    wsems[c % _NB])

        def wdrain(c):
            @pl.when(valid(c))
            def _():
                pltpu.make_async_copy(
                    bufs[c % _NB], out_hbm.at[pl.ds(row0, _CHUNK)],
                    wsems[c % _NB]).wait()

        rem0 = len_local % _CHUNK
        cb = len_local // _CHUNK

        def ztail(c, buf):
            @pl.when((rem0 > 0) & (c == cb))
            def _():
                def zrow(r, _):
                    @pl.when(r >= rem0)
                    def _():
                        for jj in range(_D // _L):
                            buf[r, pl.ds(jj * _L, _L)] = (
                                jnp.zeros((_L,), jnp.float32))
                    return 0

                lax.fori_loop(0, _CHUNK, zrow, 0)

        nk = (_ROWS_W - len_local) // _ZROWS

        def zfill(k, _):
            off = pl.multiple_of(_ROWS_W - _ZROWS * (k + 1), _ZROWS)
            pltpu.async_copy(zbuf, out_hbm.at[pl.ds(row0 + off, _ZROWS)],
                             zsem)
            return 0

        lax.fori_loop(0, nk, zfill, 0)

        for c in range(_GF):
            gstart(c)

        @pl.when(wid == 0)
        def _():
            pltpu.sync_copy(dur_hbm, dur_all_v)
            lane_base = lax.iota(jnp.int32, _L) * _S

            def len_step(i, acc):
                v = plsc.load_gather(dur_all_v, [lane_base + i])
                return acc + jnp.clip(v, 0, 300)

            tot = lax.fori_loop(0, _S, len_step, jnp.zeros((_L,), jnp.int32))
            len_v[...] = jnp.minimum(tot, ml_v[...])
            pltpu.sync_copy(len_v, len_hbm)

        for c in range(_NCHUNK):
            gwait(c)
            if c + _GF < _NCHUNK:
                if c >= _NB - _GF:
                    wdrain(c - (_NB - _GF))
                gstart(c + _GF)
            ztail(c, bufs[c % _NB])
            wstart(c)
        for c in range(max(0, _NCHUNK - _NB), _NCHUNK):
            wdrain(c)

        def zdrain(k, _):
            pltpu.make_async_copy(
                zbuf, out_hbm.at[pl.ds(row0, _ZROWS)], zsem).wait()
            return 0

        lax.fori_loop(0, nk, zdrain, 0)

    return expand


_EXPAND = _build_expand()


def kernel(x, duration, max_len):
    x_flat = x.reshape(_B * _S, _D)
    dur_flat = duration.astype(jnp.int32).reshape(_B * _S)
    zc = jnp.zeros((_ZROWS, _D), jnp.float32)
    ml = jnp.full((_L,), max_len, jnp.int32)
    out_flat, lengths = _EXPAND(dur_flat, x_flat, zc, ml)
    return out_flat.reshape(_B, _T, _D), lengths


# speedup vs baseline: 1.0231x; 1.0231x over previous
import functools

import jax
import jax.numpy as jnp
from jax import lax
from jax.experimental import pallas as pl
from jax.experimental.pallas import tpu as pltpu
from jax.experimental.pallas import tpu_sc as plsc

_B, _S, _D, _T = 16, 512, 512, 2048
_L = 16
_NW = 32
_ROWS_W = (_B * _T) // _NW
_CHUNK = 64
_NCHUNK = _ROWS_W // _CHUNK
_ZROWS = 32
_NB = 3
_GF = 2


def _build_expand():
    mesh = plsc.VectorSubcoreMesh(core_axis_name="c", subcore_axis_name="s")

    @functools.partial(
        pl.kernel,
        mesh=mesh,
        compiler_params=pltpu.CompilerParams(needs_layout_passes=False),
        out_type=(
            jax.ShapeDtypeStruct((_B * _T, _D), jnp.float32),
            jax.ShapeDtypeStruct((_B,), jnp.int32),
        ),
        scratch_types=[
            pltpu.VMEM((_S,), jnp.int32),
            pltpu.VMEM((_S,), jnp.int32),
            pltpu.VMEM((_ROWS_W,), jnp.int32),
            *[pltpu.VMEM((_CHUNK, _D), jnp.float32) for _ in range(_NB)],
            pltpu.VMEM((_ZROWS, _D), jnp.float32),
            pltpu.VMEM((_B * _S,), jnp.int32),
            pltpu.VMEM((_B,), jnp.int32),
            pltpu.VMEM((_L,), jnp.int32),
            *[pltpu.SemaphoreType.DMA for _ in range(_NB)],
            *[pltpu.SemaphoreType.DMA for _ in range(_NB)],
            pltpu.SemaphoreType.DMA,
        ],
    )
    def expand(dur_hbm, xflat_hbm, zc_hbm, ml_hbm, out_hbm, len_hbm,
               *refs):
        bufs = refs[3:3 + _NB]
        (dur_v, cum_v, gidx_v) = refs[:3]
        (zbuf, dur_all_v, len_v, ml_v) = refs[3 + _NB:7 + _NB]
        gsems = refs[7 + _NB:7 + 2 * _NB]
        wsems = refs[7 + 2 * _NB:7 + 3 * _NB]
        zsem = refs[7 + 3 * _NB]
        cid = lax.axis_index("c")
        sid = lax.axis_index("s")
        wid = cid * (_NW // 2) + sid
        b = wid // 2
        t0 = (wid % 2) * (_T // 2)

        pltpu.sync_copy(dur_hbm.at[pl.ds(b * _S, _S)], dur_v)
        pltpu.sync_copy(ml_hbm, ml_v)
        pltpu.sync_copy(zc_hbm, zbuf)
        max_len = ml_v[...][0]

        def cum_step(i, carry):
            v = jnp.clip(dur_v[pl.ds(i * _L, _L)], 0, 300)
            cum_v[pl.ds(i * _L, _L)] = plsc.cumsum(v) + carry
            return carry + jnp.sum(v)

        total = lax.fori_loop(0, _S // _L, cum_step, jnp.int32(0))
        len_b = jnp.minimum(total, max_len)
        len_local = jnp.clip(len_b - t0, 0, _ROWS_W)

        def idx_step(i, _):
            t = t0 + i * _L + lax.iota(jnp.int32, _L)
            j = jnp.zeros((_L,), jnp.int32)
            for s in (256, 128, 64, 32, 16, 8, 4, 2, 1):
                m = j + s
                probe = plsc.load_gather(cum_v, [jnp.minimum(m - 1, _S - 1)])
                j = jnp.where(probe <= t, m, j)
            tok = jnp.where(t < len_b, jnp.minimum(j, _S - 1), 0)
            gidx_v[pl.ds(i * _L, _L)] = b * _S + tok
            return 0

        row0 = wid * _ROWS_W

        def valid(c):
            return c * _CHUNK < len_local

        def gstart(c):
            @pl.when(valid(c))
            def _():
                pltpu.async_copy(
                    xflat_hbm.at[gidx_v.at[pl.ds(c * _CHUNK, _CHUNK)]],
                    bufs[c % _NB], gsems[c % _NB])

        def gwait(c):
            @pl.when(valid(c))
            def _():
                pltpu.make_async_copy(
                    xflat_hbm.at[pl.ds(0, _CHUNK)], bufs[c % _NB],
                    gsems[c % _NB]).wait()

        def wstart(c):
            @pl.when(valid(c))
            def _():
                pltpu.async_copy(
                    bufs[c % _NB], out_hbm.at[pl.ds(row0 + c * _CHUNK, _CHUNK)],
                    wsems[c % _NB])

        def wdrain(c):
            @pl.when(valid(c))
            def _():
                pltpu.make_async_copy(
                    bufs[c % _NB], out_hbm.at[pl.ds(row0, _CHUNK)],
                    wsems[c % _NB]).wait()

        rem0 = len_local % _CHUNK
        cb = len_local // _CHUNK

        def ztail(c, buf):
            @pl.when((rem0 > 0) & (c == cb))
            def _():
                def zrow(r, _):
                    @pl.when(r >= rem0)
                    def _():
                        for jj in range(_D // _L):
                            buf[r, pl.ds(jj * _L, _L)] = (
                                jnp.zeros((_L,), jnp.float32))
                    return 0

                lax.fori_loop(0, _CHUNK, zrow, 0)

        nk = (_ROWS_W - len_local) // _ZROWS

        def zfill(k, _):
            off = pl.multiple_of(_ROWS_W - _ZROWS * (k + 1), _ZROWS)
            pltpu.async_copy(zbuf, out_hbm.at[pl.ds(row0 + off, _ZROWS)],
                             zsem)
            return 0

        lax.fori_loop(0, nk, zfill, 0)

        _PRE = _GF * _CHUNK // _L
        lax.fori_loop(0, _PRE, idx_step, 0)
        for c in range(_GF):
            gstart(c)
        lax.fori_loop(_PRE, _ROWS_W // _L, idx_step, 0)

        @pl.when(wid == 0)
        def _():
            pltpu.sync_copy(dur_hbm, dur_all_v)
            lane_base = lax.iota(jnp.int32, _L) * _S

            def len_step(i, acc):
                v = plsc.load_gather(dur_all_v, [lane_base + i])
                return acc + jnp.clip(v, 0, 300)

            tot = lax.fori_loop(0, _S, len_step, jnp.zeros((_L,), jnp.int32))
            len_v[...] = jnp.minimum(tot, ml_v[...])
            pltpu.sync_copy(len_v, len_hbm)

        for c in range(_NCHUNK):
            gwait(c)
            if c + _GF < _NCHUNK:
                if c >= _NB - _GF:
                    wdrain(c - (_NB - _GF))
                gstart(c + _GF)
            ztail(c, bufs[c % _NB])
            wstart(c)
        for c in range(max(0, _NCHUNK - _NB), _NCHUNK):
            wdrain(c)

        def zdrain(k, _):
            pltpu.make_async_copy(
                zbuf, out_hbm.at[pl.ds(row0, _ZROWS)], zsem).wait()
            return 0

        lax.fori_loop(0, nk, zdrain, 0)

    return expand


_EXPAND = _build_expand()


def kernel(x, duration, max_len):
    x_flat = x.reshape(_B * _S, _D)
    dur_flat = duration.astype(jnp.int32).reshape(_B * _S)
    zc = jnp.zeros((_ZROWS, _D), jnp.float32)
    ml = jnp.full((_L,), max_len, jnp.int32)
    out_flat, lengths = _EXPAND(dur_flat, x_flat, zc, ml)
    return out_flat.reshape(_B, _T, _D), lengths


# speedup vs baseline: 1.0280x; 1.0048x over previous
import functools

import jax
import jax.numpy as jnp
from jax import lax
from jax.experimental import pallas as pl
from jax.experimental.pallas import tpu as pltpu
from jax.experimental.pallas import tpu_sc as plsc

_B, _S, _D, _T = 16, 512, 512, 2048
_L = 16
_NW = 32
_ROWS_W = (_B * _T) // _NW
_CHUNK = 64
_NCHUNK = _ROWS_W // _CHUNK
_ZROWS = 32
_NB = 3
_GF = 2


def _build_expand():
    mesh = plsc.VectorSubcoreMesh(core_axis_name="c", subcore_axis_name="s")

    @functools.partial(
        pl.kernel,
        mesh=mesh,
        compiler_params=pltpu.CompilerParams(needs_layout_passes=False),
        out_type=(
            jax.ShapeDtypeStruct((_B * _T, _D), jnp.float32),
            jax.ShapeDtypeStruct((_B,), jnp.int32),
        ),
        scratch_types=[
            pltpu.VMEM((_S,), jnp.int32),
            pltpu.VMEM((_S,), jnp.int32),
            pltpu.VMEM((_ROWS_W,), jnp.int32),
            *[pltpu.VMEM((_CHUNK, _D), jnp.float32) for _ in range(_NB)],
            pltpu.VMEM((_ZROWS, _D), jnp.float32),
            pltpu.VMEM((_B * _S,), jnp.int32),
            pltpu.VMEM((_B,), jnp.int32),
            pltpu.VMEM((_L,), jnp.int32),
            *[pltpu.SemaphoreType.DMA for _ in range(_NB)],
            *[pltpu.SemaphoreType.DMA for _ in range(_NB)],
            pltpu.SemaphoreType.DMA,
        ],
    )
    def expand(dur_hbm, xflat_hbm, zc_hbm, ml_hbm, out_hbm, len_hbm,
               *refs):
        bufs = refs[3:3 + _NB]
        (dur_v, cum_v, gidx_v) = refs[:3]
        (zbuf, dur_all_v, len_v, ml_v) = refs[3 + _NB:7 + _NB]
        gsems = refs[7 + _NB:7 + 2 * _NB]
        wsems = refs[7 + 2 * _NB:7 + 3 * _NB]
        zsem = refs[7 + 3 * _NB]
        cid = lax.axis_index("c")
        sid = lax.axis_index("s")
        wid = cid * (_NW // 2) + sid
        b = wid // 2
        t0 = (wid % 2) * (_T // 2)

        pltpu.sync_copy(dur_hbm.at[pl.ds(b * _S, _S)], dur_v)
        pltpu.sync_copy(ml_hbm, ml_v)
        pltpu.sync_copy(zc_hbm, zbuf)
        max_len = ml_v[...][0]

        def cum_step(i, carry):
            v = jnp.clip(dur_v[pl.ds(i * _L, _L)], 0, 300)
            cum_v[pl.ds(i * _L, _L)] = plsc.cumsum(v) + carry
            return carry + jnp.sum(v)

        total = lax.fori_loop(0, _S // _L, cum_step, jnp.int32(0))
        len_b = jnp.minimum(total, max_len)
        len_local = jnp.clip(len_b - t0, 0, _ROWS_W)

        def idx_step(i, _):
            t = t0 + i * _L + lax.iota(jnp.int32, _L)
            j = jnp.zeros((_L,), jnp.int32)
            for s in (256, 128, 64, 32, 16, 8, 4, 2, 1):
                m = j + s
                probe = plsc.load_gather(cum_v, [m - 1])
                j = jnp.where(probe <= t, m, j)
            tok = jnp.where(t < len_b, j, 0)
            gidx_v[pl.ds(i * _L, _L)] = b * _S + tok
            return 0

        row0 = wid * _ROWS_W

        def valid(c):
            return c * _CHUNK < len_local

        def gstart(c):
            @pl.when(valid(c))
            def _():
                pltpu.async_copy(
                    xflat_hbm.at[gidx_v.at[pl.ds(c * _CHUNK, _CHUNK)]],
                    bufs[c % _NB], gsems[c % _NB])

        def gwait(c):
            @pl.when(valid(c))
            def _():
                pltpu.make_async_copy(
                    xflat_hbm.at[pl.ds(0, _CHUNK)], bufs[c % _NB],
                    gsems[c % _NB]).wait()

        def wstart(c):
            @pl.when(valid(c))
            def _():
                pltpu.async_copy(
                    bufs[c % _NB], out_hbm.at[pl.ds(row0 + c * _CHUNK, _CHUNK)],
                    wsems[c % _NB])

        def wdrain(c):
            @pl.when(valid(c))
            def _():
                pltpu.make_async_copy(
                    bufs[c % _NB], out_hbm.at[pl.ds(row0, _CHUNK)],
                    wsems[c % _NB]).wait()

        rem0 = len_local % _CHUNK
        cb = len_local // _CHUNK

        def ztail(c, buf):
            @pl.when((rem0 > 0) & (c == cb))
            def _():
                def zrow(r, _):
                    @pl.when(r >= rem0)
                    def _():
                        for jj in range(_D // _L):
                            buf[r, pl.ds(jj * _L, _L)] = (
                                jnp.zeros((_L,), jnp.float32))
                    return 0

                lax.fori_loop(0, _CHUNK, zrow, 0)

        nk = (_ROWS_W - len_local) // _ZROWS

        def zfill(k, _):
            off = pl.multiple_of(_ROWS_W - _ZROWS * (k + 1), _ZROWS)
            pltpu.async_copy(zbuf, out_hbm.at[pl.ds(row0 + off, _ZROWS)],
                             zsem)
            return 0

        lax.fori_loop(0, nk, zfill, 0)

        _PRE = _GF * _CHUNK // _L
        lax.fori_loop(0, _PRE, idx_step, 0)
        for c in range(_GF):
            gstart(c)
        lax.fori_loop(_PRE, _ROWS_W // _L, idx_step, 0)

        @pl.when(wid == 0)
        def _():
            pltpu.sync_copy(dur_hbm, dur_all_v)
            lane_base = lax.iota(jnp.int32, _L) * _S

            def len_step(i, acc):
                v = plsc.load_gather(dur_all_v, [lane_base + i])
                return acc + jnp.clip(v, 0, 300)

            tot = lax.fori_loop(0, _S, len_step, jnp.zeros((_L,), jnp.int32))
            len_v[...] = jnp.minimum(tot, ml_v[...])
            pltpu.sync_copy(len_v, len_hbm)

        for c in range(_NCHUNK):
            gwait(c)
            if c + _GF < _NCHUNK:
                if c >= _NB - _GF:
                    wdrain(c - (_NB - _GF))
                gstart(c + _GF)
            ztail(c, bufs[c % _NB])
            wstart(c)
        for c in range(max(0, _NCHUNK - _NB), _NCHUNK):
            wdrain(c)

        def zdrain(k, _):
            pltpu.make_async_copy(
                zbuf, out_hbm.at[pl.ds(row0, _ZROWS)], zsem).wait()
            return 0

        lax.fori_loop(0, nk, zdrain, 0)

    return expand


_EXPAND = _build_expand()


def kernel(x, duration, max_len):
    x_flat = x.reshape(_B * _S, _D)
    dur_flat = duration.astype(jnp.int32).reshape(_B * _S)
    zc = jnp.zeros((_ZROWS, _D), jnp.float32)
    ml = jnp.full((_L,), max_len, jnp.int32)
    out_flat, lengths = _EXPAND(dur_flat, x_flat, zc, ml)
    return out_flat.reshape(_B, _T, _D), lengths
